# baseline (speedup 1.0000x reference)
import functools

import jax
import jax.numpy as jnp
from jax import lax
from jax.experimental import pallas as pl
from jax.experimental.pallas import tpu as pltpu
from jax.experimental.pallas import tpu_sc as plsc

N_NODES = 10000
D_IN = 128
HIDDEN = 128
EMBED = 64
NEG_SLOPE = 0.01

N_EDGES = 320000
NUM_CORES = 2
NUM_SUBCORES = 16
NW = NUM_CORES * NUM_SUBCORES
CHUNK = 128
DEG_NCHUNK = 80
MP_NCHUNK = 160
E_PAD = CHUNK * DEG_NCHUNK * NW
N_ACC = 10240
ROWS_PER_TILE = N_ACC // NUM_SUBCORES
DEG_W = 16
NBUF = 4


def _deg_body(dst_hbm, ones_hbm, zeros_hbm, out_hbm, didx, ones_v, stage, col, acc):
    c = lax.axis_index("c")
    s = lax.axis_index("s")
    w = s * NUM_CORES + c
    pltpu.sync_copy(zeros_hbm, stage)
    pltpu.sync_copy(stage, acc.at[pl.ds(s * ROWS_PER_TILE, ROWS_PER_TILE)])
    plsc.subcore_barrier()
    pltpu.sync_copy(dst_hbm.at[w], didx)
    pltpu.sync_copy(ones_hbm, ones_v)

    def body(j, carry):
        pltpu.sync_copy(ones_v, acc.at[didx.at[j]], add=True)
        return carry

    lax.fori_loop(0, DEG_NCHUNK, body, 0)
    plsc.subcore_barrier()
    sl = pl.ds(s * ROWS_PER_TILE, ROWS_PER_TILE)
    pltpu.sync_copy(acc.at[sl, pl.ds(0, 1)], col)
    pltpu.sync_copy(col, out_hbm.at[c, sl])


_deg_call = functools.partial(
    pl.kernel,
    out_type=jax.ShapeDtypeStruct((NUM_CORES, N_ACC, 1), jnp.float32),
    mesh=plsc.VectorSubcoreMesh(core_axis_name="c", subcore_axis_name="s"),
    compiler_params=pltpu.CompilerParams(use_tc_tiling_on_sc=False),
    scratch_types=[
        pltpu.VMEM((DEG_NCHUNK, CHUNK), jnp.int32),
        pltpu.VMEM((CHUNK, DEG_W), jnp.float32),
        pltpu.VMEM((ROWS_PER_TILE, DEG_W), jnp.float32),
        pltpu.VMEM((ROWS_PER_TILE, 1), jnp.float32),
        pltpu.VMEM_SHARED((N_ACC, DEG_W), jnp.float32),
    ],
)(_deg_body)


def _mp_body(rows_hbm, src_hbm, dst_hbm, zeros_hbm, out_hbm,
             sidx, didx, r0, r1, r2, r3, stage, acc,
             g0, g1, g2, g3, t0, t1, t2, t3):
    c = lax.axis_index("c")
    s = lax.axis_index("s")
    rows = (r0, r1, r2, r3)
    gsem = (g0, g1, g2, g3)
    ssem = (t0, t1, t2, t3)
    pltpu.sync_copy(zeros_hbm, stage)
    for z in range(ROWS_PER_TILE // CHUNK):
        pltpu.sync_copy(stage, acc.at[pl.ds(s * ROWS_PER_TILE + z * CHUNK, CHUNK)])
    plsc.subcore_barrier()
    pltpu.sync_copy(src_hbm.at[c, s], sidx)
    pltpu.sync_copy(dst_hbm.at[s], didx)

    pltpu.async_copy(rows_hbm.at[sidx.at[0]], rows[0], gsem[0])
    pltpu.async_copy(rows_hbm.at[sidx.at[1]], rows[1], gsem[1])

    ngroup = MP_NCHUNK // NBUF

    def group(g, carry):
        for q in range(NBUF):
            j = NBUF * g + q
            qn = (q + 2) % NBUF
            pltpu.make_async_copy(rows_hbm.at[sidx.at[j]], rows[q], gsem[q]).wait()
            pltpu.async_copy(rows[q], acc.at[didx.at[j]], ssem[q], add=True)

            if q >= 2:
                pltpu.make_async_copy(
                    rows[qn], acc.at[didx.at[j - 2]], ssem[qn]).wait()
                @pl.when(g < ngroup - 1)
                def _():
                    pltpu.async_copy(rows_hbm.at[sidx.at[j + 2]], rows[qn], gsem[qn])
            else:
                @pl.when(g > 0)
                def _():
                    pltpu.make_async_copy(
                        rows[qn], acc.at[didx.at[j - 2]], ssem[qn]).wait()
                pltpu.async_copy(rows_hbm.at[sidx.at[j + 2]], rows[qn], gsem[qn])
        return carry

    lax.fori_loop(0, ngroup, group, 0)
    pltpu.make_async_copy(rows[2], acc.at[didx.at[MP_NCHUNK - 2]], ssem[2]).wait()
    pltpu.make_async_copy(rows[3], acc.at[didx.at[MP_NCHUNK - 1]], ssem[3]).wait()
    plsc.subcore_barrier()
    for z in range(ROWS_PER_TILE // CHUNK):
        sl = pl.ds(s * ROWS_PER_TILE + z * CHUNK, CHUNK)
        pltpu.sync_copy(acc.at[sl], stage)
        pltpu.sync_copy(stage, out_hbm.at[c, sl])


def _make_mp_call(dh):
    return functools.partial(
        pl.kernel,
        out_type=jax.ShapeDtypeStruct((NUM_CORES, N_ACC, dh), jnp.float32),
        mesh=plsc.VectorSubcoreMesh(core_axis_name="c", subcore_axis_name="s"),
        compiler_params=pltpu.CompilerParams(use_tc_tiling_on_sc=False),
        scratch_types=[
            pltpu.VMEM((MP_NCHUNK, CHUNK), jnp.int32),
            pltpu.VMEM((MP_NCHUNK, CHUNK), jnp.int32),
            pltpu.VMEM((CHUNK, dh), jnp.float32),
            pltpu.VMEM((CHUNK, dh), jnp.float32),
            pltpu.VMEM((CHUNK, dh), jnp.float32),
            pltpu.VMEM((CHUNK, dh), jnp.float32),
            pltpu.VMEM((CHUNK, dh), jnp.float32),
            pltpu.VMEM_SHARED((N_ACC, dh), jnp.float32),
            pltpu.SemaphoreType.DMA,
            pltpu.SemaphoreType.DMA,
            pltpu.SemaphoreType.DMA,
            pltpu.SemaphoreType.DMA,
            pltpu.SemaphoreType.DMA,
            pltpu.SemaphoreType.DMA,
            pltpu.SemaphoreType.DMA,
            pltpu.SemaphoreType.DMA,
        ],
    )(_mp_body)


_mp_call_64 = _make_mp_call(HIDDEN // 2)
_mp_call_32 = _make_mp_call(EMBED // 2)


_BLK = 1000
_GRID = N_NODES // _BLK


def _dinv_from(deg_ref):
    d = deg_ref[0] + deg_ref[1]
    return lax.rsqrt(d + 1.0)


def _tc1_body(deg_ref, x_ref, w1_ref, o_ref):
    dinv = _dinv_from(deg_ref)
    xw = jnp.dot(x_ref[...], w1_ref[...], preferred_element_type=jnp.float32)
    yw = xw * dinv
    o_ref[0] = yw[:, : HIDDEN // 2]
    o_ref[1] = yw[:, HIDDEN // 2 :]


def _tc2_body(deg_ref, acc_ref, yw_ref, b1_ref, w2_ref, o_ref):
    dinv = _dinv_from(deg_ref)
    a = jnp.concatenate([acc_ref[0], acc_ref[1]], axis=1)
    y = jnp.concatenate([yw_ref[0], yw_ref[1]], axis=1)
    h = (a + y) * dinv + b1_ref[...]
    h = jnp.where(h > 0, h, NEG_SLOPE * h)
    yw2 = jnp.dot(h, w2_ref[...], preferred_element_type=jnp.float32) * dinv
    o_ref[0] = yw2[:, : EMBED // 2]
    o_ref[1] = yw2[:, EMBED // 2 :]


def _tc3_body(deg_ref, acc_ref, yw_ref, b2_ref, w3_ref, b3_ref, o_ref):
    dinv = _dinv_from(deg_ref)
    a = jnp.concatenate([acc_ref[0], acc_ref[1]], axis=1)
    y = jnp.concatenate([yw_ref[0], yw_ref[1]], axis=1)
    h = (a + y) * dinv + b2_ref[...]
    h = jnp.where(h > 0, h, NEG_SLOPE * h)
    o_ref[...] = jnp.dot(h, w3_ref[...], preferred_element_type=jnp.float32) + b3_ref[0, 0]


def _deg_spec():
    return pl.BlockSpec((NUM_CORES, _BLK, 1), lambda i: (0, i, 0))


def _full(shape):
    return pl.BlockSpec(shape, lambda i: tuple(0 for _ in shape))


def _rows(d):
    return pl.BlockSpec((_BLK, d), lambda i: (i, 0))


def _split_spec(dh):
    return pl.BlockSpec((NUM_CORES, _BLK, dh), lambda i: (0, i, 0))


def kernel(x, edge_index, W1, b1, W2, b2, W3, b3):
    src = edge_index[0].astype(jnp.int32)
    dst = edge_index[1].astype(jnp.int32)
    pad = E_PAD - N_EDGES
    srcp = jnp.concatenate([src, jnp.zeros((pad,), jnp.int32)])
    dstp = jnp.concatenate([dst, jnp.full((pad,), N_NODES, jnp.int32)])
    dst_deg = dstp.reshape(NW, DEG_NCHUNK, CHUNK)
    src_mp = jnp.stack([srcp, srcp + N_NODES]).reshape(
        NUM_CORES, NUM_SUBCORES, MP_NCHUNK, CHUNK)
    dst_mp = dstp.reshape(NUM_SUBCORES, MP_NCHUNK, CHUNK)

    ones_deg = jnp.ones((CHUNK, DEG_W), jnp.float32)
    zeros_deg = jnp.zeros((ROWS_PER_TILE, DEG_W), jnp.float32)
    zeros64 = jnp.zeros((CHUNK, HIDDEN // 2), jnp.float32)
    zeros32 = jnp.zeros((CHUNK, EMBED // 2), jnp.float32)

    deg = _deg_call(dst_deg, ones_deg, zeros_deg)

    yw1 = pl.pallas_call(
        _tc1_body,
        grid=(_GRID,),
        in_specs=[_deg_spec(), _rows(D_IN), _full((D_IN, HIDDEN))],
        out_specs=_split_spec(HIDDEN // 2),
        out_shape=jax.ShapeDtypeStruct((NUM_CORES, N_NODES, HIDDEN // 2), jnp.float32),
    )(deg, x, W1)

    acc1 = _mp_call_64(
        yw1.reshape(NUM_CORES * N_NODES, HIDDEN // 2), src_mp, dst_mp, zeros64)

    yw2 = pl.pallas_call(
        _tc2_body,
        grid=(_GRID,),
        in_specs=[_deg_spec(), _split_spec(HIDDEN // 2), _split_spec(HIDDEN // 2),
                  _full((HIDDEN,)), _full((HIDDEN, EMBED))],
        out_specs=_split_spec(EMBED // 2),
        out_shape=jax.ShapeDtypeStruct((NUM_CORES, N_NODES, EMBED // 2), jnp.float32),
    )(deg, acc1, yw1, b1, W2)

    acc2 = _mp_call_32(
        yw2.reshape(NUM_CORES * N_NODES, EMBED // 2), src_mp, dst_mp, zeros32)

    out = pl.pallas_call(
        _tc3_body,
        grid=(_GRID,),
        in_specs=[_deg_spec(), _split_spec(EMBED // 2), _split_spec(EMBED // 2),
                  _full((EMBED,)), _full((EMBED, 1)),
                  pl.BlockSpec(memory_space=pltpu.SMEM)],
        out_specs=_rows(1),
        out_shape=jax.ShapeDtypeStruct((N_NODES, 1), jnp.float32),
    )(deg, acc2, yw2, b2, W3, b3.reshape(1, 1))

    return out.reshape(-1)


# speedup vs baseline: 1.0464x; 1.0464x over previous
import functools

import jax
import jax.numpy as jnp
from jax import lax
from jax.experimental import pallas as pl
from jax.experimental.pallas import tpu as pltpu
from jax.experimental.pallas import tpu_sc as plsc

N_NODES = 10000
D_IN = 128
HIDDEN = 128
EMBED = 64
NEG_SLOPE = 0.01

N_EDGES = 320000
NUM_CORES = 2
NUM_SUBCORES = 16
NW = NUM_CORES * NUM_SUBCORES
CHUNK = 128
DEG_NCHUNK = 80
MP_NCHUNK = 160
E_PAD = CHUNK * DEG_NCHUNK * NW
N_ACC = 10240
ROWS_PER_TILE = N_ACC // NUM_SUBCORES
DEG_W = 16
NBUF = 5


def _deg_body(dst_hbm, ones_hbm, zeros_hbm, out_hbm, didx, ones_v, stage, col, acc):
    c = lax.axis_index("c")
    s = lax.axis_index("s")
    w = s * NUM_CORES + c
    pltpu.sync_copy(zeros_hbm, stage)
    pltpu.sync_copy(stage, acc.at[pl.ds(s * ROWS_PER_TILE, ROWS_PER_TILE)])
    plsc.subcore_barrier()
    pltpu.sync_copy(dst_hbm.at[w], didx)
    pltpu.sync_copy(ones_hbm, ones_v)

    def body(j, carry):
        pltpu.sync_copy(ones_v, acc.at[didx.at[j]], add=True)
        return carry

    lax.fori_loop(0, DEG_NCHUNK, body, 0)
    plsc.subcore_barrier()
    sl = pl.ds(s * ROWS_PER_TILE, ROWS_PER_TILE)
    pltpu.sync_copy(acc.at[sl, pl.ds(0, 1)], col)
    pltpu.sync_copy(col, out_hbm.at[c, sl])


_deg_call = functools.partial(
    pl.kernel,
    out_type=jax.ShapeDtypeStruct((NUM_CORES, N_ACC, 1), jnp.float32),
    mesh=plsc.VectorSubcoreMesh(core_axis_name="c", subcore_axis_name="s"),
    compiler_params=pltpu.CompilerParams(use_tc_tiling_on_sc=False),
    scratch_types=[
        pltpu.VMEM((DEG_NCHUNK, CHUNK), jnp.int32),
        pltpu.VMEM((CHUNK, DEG_W), jnp.float32),
        pltpu.VMEM((ROWS_PER_TILE, DEG_W), jnp.float32),
        pltpu.VMEM((ROWS_PER_TILE, 1), jnp.float32),
        pltpu.VMEM_SHARED((N_ACC, DEG_W), jnp.float32),
    ],
)(_deg_body)


def _mp_body(rows_hbm, src_hbm, dst_hbm, zeros_hbm, out_hbm,
             sidx, didx, r0, r1, r2, r3, r4, acc, s0, s1, s2, s3, s4):
    c = lax.axis_index("c")
    s = lax.axis_index("s")
    rows = (r0, r1, r2, r3, r4)
    sems = (s0, s1, s2, s3, s4)
    pltpu.sync_copy(zeros_hbm, r0)
    for z in range(ROWS_PER_TILE // CHUNK):
        pltpu.sync_copy(r0, acc.at[pl.ds(s * ROWS_PER_TILE + z * CHUNK, CHUNK)])
    plsc.subcore_barrier()
    pltpu.sync_copy(src_hbm.at[c, s], sidx)
    pltpu.sync_copy(dst_hbm.at[s], didx)

    for q in range(NBUF):
        pltpu.async_copy(rows_hbm.at[sidx.at[q]], rows[q], sems[q])

    ngroup = MP_NCHUNK // NBUF

    def group(g, carry):
        for q in range(NBUF):
            j = NBUF * g + q
            pltpu.make_async_copy(rows_hbm.at[sidx.at[j]], rows[q], sems[q]).wait()
            pltpu.sync_copy(rows[q], acc.at[didx.at[j]], add=True)

            @pl.when(g < ngroup - 1)
            def _():
                pltpu.async_copy(rows_hbm.at[sidx.at[j + NBUF]], rows[q], sems[q])

        return carry

    lax.fori_loop(0, ngroup, group, 0)
    plsc.subcore_barrier()
    for z in range(ROWS_PER_TILE // CHUNK):
        sl = pl.ds(s * ROWS_PER_TILE + z * CHUNK, CHUNK)
        pltpu.sync_copy(acc.at[sl], r0)
        pltpu.sync_copy(r0, out_hbm.at[c, sl])


def _make_mp_call(dh):
    return functools.partial(
        pl.kernel,
        out_type=jax.ShapeDtypeStruct((NUM_CORES, N_ACC, dh), jnp.float32),
        mesh=plsc.VectorSubcoreMesh(core_axis_name="c", subcore_axis_name="s"),
        compiler_params=pltpu.CompilerParams(use_tc_tiling_on_sc=False),
        scratch_types=[
            pltpu.VMEM((MP_NCHUNK, CHUNK), jnp.int32),
            pltpu.VMEM((MP_NCHUNK, CHUNK), jnp.int32),
            pltpu.VMEM((CHUNK, dh), jnp.float32),
            pltpu.VMEM((CHUNK, dh), jnp.float32),
            pltpu.VMEM((CHUNK, dh), jnp.float32),
            pltpu.VMEM((CHUNK, dh), jnp.float32),
            pltpu.VMEM((CHUNK, dh), jnp.float32),
            pltpu.VMEM_SHARED((N_ACC, dh), jnp.float32),
            pltpu.SemaphoreType.DMA,
            pltpu.SemaphoreType.DMA,
            pltpu.SemaphoreType.DMA,
            pltpu.SemaphoreType.DMA,
            pltpu.SemaphoreType.DMA,
        ],
    )(_mp_body)


_mp_call_64 = _make_mp_call(HIDDEN // 2)
_mp_call_32 = _make_mp_call(EMBED // 2)


_BLK = 1000
_GRID = N_NODES // _BLK


def _dinv_from(deg_ref):
    d = deg_ref[0] + deg_ref[1]
    return lax.rsqrt(d + 1.0)


def _tc1_body(deg_ref, x_ref, w1_ref, o_ref):
    dinv = _dinv_from(deg_ref)
    xw = jnp.dot(x_ref[...], w1_ref[...], preferred_element_type=jnp.float32)
    yw = xw * dinv
    o_ref[0] = yw[:, : HIDDEN // 2]
    o_ref[1] = yw[:, HIDDEN // 2 :]


def _tc2_body(deg_ref, acc_ref, yw_ref, b1_ref, w2_ref, o_ref):
    dinv = _dinv_from(deg_ref)
    a = jnp.concatenate([acc_ref[0], acc_ref[1]], axis=1)
    y = jnp.concatenate([yw_ref[0], yw_ref[1]], axis=1)
    h = (a + y) * dinv + b1_ref[...]
    h = jnp.where(h > 0, h, NEG_SLOPE * h)
    yw2 = jnp.dot(h, w2_ref[...], preferred_element_type=jnp.float32) * dinv
    o_ref[0] = yw2[:, : EMBED // 2]
    o_ref[1] = yw2[:, EMBED // 2 :]


def _tc3_body(deg_ref, acc_ref, yw_ref, b2_ref, w3_ref, b3_ref, o_ref):
    dinv = _dinv_from(deg_ref)
    a = jnp.concatenate([acc_ref[0], acc_ref[1]], axis=1)
    y = jnp.concatenate([yw_ref[0], yw_ref[1]], axis=1)
    h = (a + y) * dinv + b2_ref[...]
    h = jnp.where(h > 0, h, NEG_SLOPE * h)
    o_ref[...] = jnp.dot(h, w3_ref[...], preferred_element_type=jnp.float32) + b3_ref[0, 0]


def _deg_spec():
    return pl.BlockSpec((NUM_CORES, _BLK, 1), lambda i: (0, i, 0))


def _full(shape):
    return pl.BlockSpec(shape, lambda i: tuple(0 for _ in shape))


def _rows(d):
    return pl.BlockSpec((_BLK, d), lambda i: (i, 0))


def _split_spec(dh):
    return pl.BlockSpec((NUM_CORES, _BLK, dh), lambda i: (0, i, 0))


def kernel(x, edge_index, W1, b1, W2, b2, W3, b3):
    src = edge_index[0].astype(jnp.int32)
    dst = edge_index[1].astype(jnp.int32)
    pad = E_PAD - N_EDGES
    srcp = jnp.concatenate([src, jnp.zeros((pad,), jnp.int32)])
    dstp = jnp.concatenate([dst, jnp.full((pad,), N_NODES, jnp.int32)])
    dst_deg = dstp.reshape(NW, DEG_NCHUNK, CHUNK)
    src_mp = jnp.stack([srcp, srcp + N_NODES]).reshape(
        NUM_CORES, NUM_SUBCORES, MP_NCHUNK, CHUNK)
    dst_mp = dstp.reshape(NUM_SUBCORES, MP_NCHUNK, CHUNK)

    ones_deg = jnp.ones((CHUNK, DEG_W), jnp.float32)
    zeros_deg = jnp.zeros((ROWS_PER_TILE, DEG_W), jnp.float32)
    zeros64 = jnp.zeros((CHUNK, HIDDEN // 2), jnp.float32)
    zeros32 = jnp.zeros((CHUNK, EMBED // 2), jnp.float32)

    deg = _deg_call(dst_deg, ones_deg, zeros_deg)

    yw1 = pl.pallas_call(
        _tc1_body,
        grid=(_GRID,),
        in_specs=[_deg_spec(), _rows(D_IN), _full((D_IN, HIDDEN))],
        out_specs=_split_spec(HIDDEN // 2),
        out_shape=jax.ShapeDtypeStruct((NUM_CORES, N_NODES, HIDDEN // 2), jnp.float32),
    )(deg, x, W1)

    acc1 = _mp_call_64(
        yw1.reshape(NUM_CORES * N_NODES, HIDDEN // 2), src_mp, dst_mp, zeros64)

    yw2 = pl.pallas_call(
        _tc2_body,
        grid=(_GRID,),
        in_specs=[_deg_spec(), _split_spec(HIDDEN // 2), _split_spec(HIDDEN // 2),
                  _full((HIDDEN,)), _full((HIDDEN, EMBED))],
        out_specs=_split_spec(EMBED // 2),
        out_shape=jax.ShapeDtypeStruct((NUM_CORES, N_NODES, EMBED // 2), jnp.float32),
    )(deg, acc1, yw1, b1, W2)

    acc2 = _mp_call_32(
        yw2.reshape(NUM_CORES * N_NODES, EMBED // 2), src_mp, dst_mp, zeros32)

    out = pl.pallas_call(
        _tc3_body,
        grid=(_GRID,),
        in_specs=[_deg_spec(), _split_spec(EMBED // 2), _split_spec(EMBED // 2),
                  _full((EMBED,)), _full((EMBED, 1)),
                  pl.BlockSpec(memory_space=pltpu.SMEM)],
        out_specs=_rows(1),
        out_shape=jax.ShapeDtypeStruct((N_NODES, 1), jnp.float32),
    )(deg, acc2, yw2, b2, W3, b3.reshape(1, 1))

    return out.reshape(-1)


# speedup vs baseline: 1.0508x; 1.0043x over previous
import functools

import jax
import jax.numpy as jnp
from jax import lax
from jax.experimental import pallas as pl
from jax.experimental.pallas import tpu as pltpu
from jax.experimental.pallas import tpu_sc as plsc

N_NODES = 10000
D_IN = 128
HIDDEN = 128
EMBED = 64
NEG_SLOPE = 0.01

N_EDGES = 320000
NUM_CORES = 2
NUM_SUBCORES = 16
NW = NUM_CORES * NUM_SUBCORES
CHUNK = 128
DEG_NCHUNK = 80
MP_NCHUNK = 160
E_PAD = CHUNK * DEG_NCHUNK * NW
N_ACC = 10240
ROWS_PER_TILE = N_ACC // NUM_SUBCORES
DEG_W = 16
NBUF = 5


def _deg_body(dst_hbm, ones_hbm, zeros_hbm, out_hbm, didx, ones_v, stage, col, acc):
    c = lax.axis_index("c")
    s = lax.axis_index("s")
    w = s * NUM_CORES + c
    pltpu.sync_copy(zeros_hbm, stage)
    pltpu.sync_copy(stage, acc.at[pl.ds(s * ROWS_PER_TILE, ROWS_PER_TILE)])
    plsc.subcore_barrier()
    pltpu.sync_copy(dst_hbm.at[w], didx)
    pltpu.sync_copy(ones_hbm, ones_v)

    def body(j, carry):
        pltpu.sync_copy(ones_v, acc.at[didx.at[j]], add=True)
        return carry

    lax.fori_loop(0, DEG_NCHUNK, body, 0)
    plsc.subcore_barrier()
    sl = pl.ds(s * ROWS_PER_TILE, ROWS_PER_TILE)
    pltpu.sync_copy(acc.at[sl, pl.ds(0, 1)], col)
    pltpu.sync_copy(col, out_hbm.at[c, sl])


_deg_call = functools.partial(
    pl.kernel,
    out_type=jax.ShapeDtypeStruct((NUM_CORES, N_ACC, 1), jnp.float32),
    mesh=plsc.VectorSubcoreMesh(core_axis_name="c", subcore_axis_name="s"),
    compiler_params=pltpu.CompilerParams(use_tc_tiling_on_sc=False),
    scratch_types=[
        pltpu.VMEM((DEG_NCHUNK, CHUNK), jnp.int32),
        pltpu.VMEM((CHUNK, DEG_W), jnp.float32),
        pltpu.VMEM((ROWS_PER_TILE, DEG_W), jnp.float32),
        pltpu.VMEM((ROWS_PER_TILE, 1), jnp.float32),
        pltpu.VMEM_SHARED((N_ACC, DEG_W), jnp.float32),
    ],
)(_deg_body)


def _mp_body(rows_hbm, src_hbm, dst_hbm, zeros_hbm, out_hbm,
             sidx, didx, r0, r1, r2, r3, r4, acc, s0, s1, s2, s3, s4):
    c = lax.axis_index("c")
    s = lax.axis_index("s")
    rows = (r0, r1, r2, r3, r4)
    sems = (s0, s1, s2, s3, s4)
    pltpu.sync_copy(src_hbm.at[c, s], sidx)
    pltpu.sync_copy(dst_hbm.at[s], didx)
    for q in range(1, NBUF):
        pltpu.async_copy(rows_hbm.at[sidx.at[q - 1]], rows[q], sems[q])
    pltpu.sync_copy(zeros_hbm, r0)
    for z in range(ROWS_PER_TILE // CHUNK):
        pltpu.sync_copy(r0, acc.at[pl.ds(s * ROWS_PER_TILE + z * CHUNK, CHUNK)])
    plsc.subcore_barrier()
    pltpu.async_copy(rows_hbm.at[sidx.at[NBUF - 1]], rows[0], sems[0])

    ngroup = MP_NCHUNK // NBUF

    def group(g, carry):
        for u in range(NBUF):
            j = NBUF * g + u
            q = (u + 1) % NBUF
            pltpu.make_async_copy(rows_hbm.at[sidx.at[j]], rows[q], sems[q]).wait()
            pltpu.sync_copy(rows[q], acc.at[didx.at[j]], add=True)

            @pl.when(g < ngroup - 1)
            def _():
                pltpu.async_copy(rows_hbm.at[sidx.at[j + NBUF]], rows[q], sems[q])

        return carry

    lax.fori_loop(0, ngroup, group, 0)
    plsc.subcore_barrier()
    for z in range(ROWS_PER_TILE // CHUNK):
        sl = pl.ds(s * ROWS_PER_TILE + z * CHUNK, CHUNK)
        pltpu.async_copy(acc.at[sl], rows[z], sems[z])
    for z in range(ROWS_PER_TILE // CHUNK):
        sl = pl.ds(s * ROWS_PER_TILE + z * CHUNK, CHUNK)
        pltpu.make_async_copy(acc.at[sl], rows[z], sems[z]).wait()
        pltpu.async_copy(rows[z], out_hbm.at[c, sl], sems[z])
    for z in range(ROWS_PER_TILE // CHUNK):
        sl = pl.ds(s * ROWS_PER_TILE + z * CHUNK, CHUNK)
        pltpu.make_async_copy(rows[z], out_hbm.at[c, sl], sems[z]).wait()


def _make_mp_call(dh):
    return functools.partial(
        pl.kernel,
        out_type=jax.ShapeDtypeStruct((NUM_CORES, N_ACC, dh), jnp.float32),
        mesh=plsc.VectorSubcoreMesh(core_axis_name="c", subcore_axis_name="s"),
        compiler_params=pltpu.CompilerParams(use_tc_tiling_on_sc=False),
        scratch_types=[
            pltpu.VMEM((MP_NCHUNK, CHUNK), jnp.int32),
            pltpu.VMEM((MP_NCHUNK, CHUNK), jnp.int32),
            pltpu.VMEM((CHUNK, dh), jnp.float32),
            pltpu.VMEM((CHUNK, dh), jnp.float32),
            pltpu.VMEM((CHUNK, dh), jnp.float32),
            pltpu.VMEM((CHUNK, dh), jnp.float32),
            pltpu.VMEM((CHUNK, dh), jnp.float32),
            pltpu.VMEM_SHARED((N_ACC, dh), jnp.float32),
            pltpu.SemaphoreType.DMA,
            pltpu.SemaphoreType.DMA,
            pltpu.SemaphoreType.DMA,
            pltpu.SemaphoreType.DMA,
            pltpu.SemaphoreType.DMA,
        ],
    )(_mp_body)


_mp_call_64 = _make_mp_call(HIDDEN // 2)
_mp_call_32 = _make_mp_call(EMBED // 2)


_BLK = 1000
_GRID = N_NODES // _BLK


def _dinv_from(deg_ref):
    d = deg_ref[0] + deg_ref[1]
    return lax.rsqrt(d + 1.0)


def _tc1_body(deg_ref, x_ref, w1_ref, o_ref):
    dinv = _dinv_from(deg_ref)
    xw = jnp.dot(x_ref[...], w1_ref[...], preferred_element_type=jnp.float32)
    yw = xw * dinv
    o_ref[0] = yw[:, : HIDDEN // 2]
    o_ref[1] = yw[:, HIDDEN // 2 :]


def _tc2_body(deg_ref, acc_ref, yw_ref, b1_ref, w2_ref, o_ref):
    dinv = _dinv_from(deg_ref)
    a = jnp.concatenate([acc_ref[0], acc_ref[1]], axis=1)
    y = jnp.concatenate([yw_ref[0], yw_ref[1]], axis=1)
    h = (a + y) * dinv + b1_ref[...]
    h = jnp.where(h > 0, h, NEG_SLOPE * h)
    yw2 = jnp.dot(h, w2_ref[...], preferred_element_type=jnp.float32) * dinv
    o_ref[0] = yw2[:, : EMBED // 2]
    o_ref[1] = yw2[:, EMBED // 2 :]


def _tc3_body(deg_ref, acc_ref, yw_ref, b2_ref, w3_ref, b3_ref, o_ref):
    dinv = _dinv_from(deg_ref)
    a = jnp.concatenate([acc_ref[0], acc_ref[1]], axis=1)
    y = jnp.concatenate([yw_ref[0], yw_ref[1]], axis=1)
    h = (a + y) * dinv + b2_ref[...]
    h = jnp.where(h > 0, h, NEG_SLOPE * h)
    o_ref[...] = jnp.dot(h, w3_ref[...], preferred_element_type=jnp.float32) + b3_ref[0, 0]


def _deg_spec():
    return pl.BlockSpec((NUM_CORES, _BLK, 1), lambda i: (0, i, 0))


def _full(shape):
    return pl.BlockSpec(shape, lambda i: tuple(0 for _ in shape))


def _rows(d):
    return pl.BlockSpec((_BLK, d), lambda i: (i, 0))


def _split_spec(dh):
    return pl.BlockSpec((NUM_CORES, _BLK, dh), lambda i: (0, i, 0))


def kernel(x, edge_index, W1, b1, W2, b2, W3, b3):
    src = edge_index[0].astype(jnp.int32)
    dst = edge_index[1].astype(jnp.int32)
    pad = E_PAD - N_EDGES
    srcp = jnp.concatenate([src, jnp.zeros((pad,), jnp.int32)])
    dstp = jnp.concatenate([dst, jnp.full((pad,), N_NODES, jnp.int32)])
    dst_deg = dstp.reshape(NW, DEG_NCHUNK, CHUNK)
    src_mp = jnp.stack([srcp, srcp + N_NODES]).reshape(
        NUM_CORES, NUM_SUBCORES, MP_NCHUNK, CHUNK)
    dst_mp = dstp.reshape(NUM_SUBCORES, MP_NCHUNK, CHUNK)

    ones_deg = jnp.ones((CHUNK, DEG_W), jnp.float32)
    zeros_deg = jnp.zeros((ROWS_PER_TILE, DEG_W), jnp.float32)
    zeros64 = jnp.zeros((CHUNK, HIDDEN // 2), jnp.float32)
    zeros32 = jnp.zeros((CHUNK, EMBED // 2), jnp.float32)

    deg = _deg_call(dst_deg, ones_deg, zeros_deg)

    yw1 = pl.pallas_call(
        _tc1_body,
        grid=(_GRID,),
        in_specs=[_deg_spec(), _rows(D_IN), _full((D_IN, HIDDEN))],
        out_specs=_split_spec(HIDDEN // 2),
        out_shape=jax.ShapeDtypeStruct((NUM_CORES, N_NODES, HIDDEN // 2), jnp.float32),
    )(deg, x, W1)

    acc1 = _mp_call_64(
        yw1.reshape(NUM_CORES * N_NODES, HIDDEN // 2), src_mp, dst_mp, zeros64)

    yw2 = pl.pallas_call(
        _tc2_body,
        grid=(_GRID,),
        in_specs=[_deg_spec(), _split_spec(HIDDEN // 2), _split_spec(HIDDEN // 2),
                  _full((HIDDEN,)), _full((HIDDEN, EMBED))],
        out_specs=_split_spec(EMBED // 2),
        out_shape=jax.ShapeDtypeStruct((NUM_CORES, N_NODES, EMBED // 2), jnp.float32),
    )(deg, acc1, yw1, b1, W2)

    acc2 = _mp_call_32(
        yw2.reshape(NUM_CORES * N_NODES, EMBED // 2), src_mp, dst_mp, zeros32)

    out = pl.pallas_call(
        _tc3_body,
        grid=(_GRID,),
        in_specs=[_deg_spec(), _split_spec(EMBED // 2), _split_spec(EMBED // 2),
                  _full((EMBED,)), _full((EMBED, 1)),
                  pl.BlockSpec(memory_space=pltpu.SMEM)],
        out_specs=_rows(1),
        out_shape=jax.ShapeDtypeStruct((N_NODES, 1), jnp.float32),
    )(deg, acc2, yw2, b2, W3, b3.reshape(1, 1))

    return out.reshape(-1)
